# Initial kernel scaffold; baseline (speedup 1.0000x reference)
import jax
import jax.numpy as jnp
from jax.experimental import pallas as pl


def kernel(x, edge_index, edge_attr, y, W1, b1, W2, b2):
    raise NotImplementedError("write your pallas kernel here")


# speedup vs baseline: 13.9458x; 13.9458x over previous
import functools

import jax
import jax.numpy as jnp
from jax import lax
from jax.experimental import pallas as pl
from jax.experimental.pallas import tpu as pltpu
from jax.experimental.pallas import tpu_sc as plsc

_N = 10000
_E = 320000
_D = 128
_NP = 10240
_CH = 128
_WT = 88
_NCH = 81
_WG = 8
_EP = 32 * _WT * _CH
_NB = 2048
_NG = _NP // _NB

_mesh = plsc.VectorSubcoreMesh(core_axis_name="c", subcore_axis_name="s")
_params = pltpu.CompilerParams(needs_layout_passes=False)


def _z16():
    return jnp.zeros((16,), jnp.float32)


@functools.partial(
    pl.kernel,
    mesh=_mesh,
    compiler_params=_params,
    out_type=jax.ShapeDtypeStruct((2, _NP), jnp.float32),
    scratch_types=[
        pltpu.VMEM((_WT, _CH), jnp.int32),
        pltpu.VMEM((_WT, _CH), jnp.float32),
        pltpu.VMEM((640,), jnp.float32),
        pltpu.VMEM_SHARED((_NP,), jnp.float32),
    ],
)
def _deg_sc(dstm, ewm, out, dstv, ewv, zbuf, dacc):
    c = lax.axis_index("c")
    s = lax.axis_index("s")
    w = c * 16 + s
    for i in range(40):
        zbuf[pl.ds(16 * i, 16)] = _z16()
    pltpu.sync_copy(zbuf, dacc.at[pl.ds(s * 640, 640)])
    plsc.subcore_barrier()
    pltpu.sync_copy(dstm.at[w], dstv)
    pltpu.sync_copy(ewm.at[w], ewv)

    def chunk(j, carry):
        pltpu.sync_copy(ewv.at[j], dacc.at[dstv.at[j]], add=True)
        return carry

    lax.fori_loop(0, _NCH, chunk, 0)
    plsc.subcore_barrier()
    pltpu.sync_copy(dacc.at[pl.ds(s * 640, 640)], out.at[c, pl.ds(s * 640, 640)])


@functools.partial(
    pl.kernel,
    mesh=_mesh,
    compiler_params=_params,
    out_type=jax.ShapeDtypeStruct((2, _NP, _D), jnp.float32),
    scratch_types=[
        pltpu.VMEM((_WG, _CH), jnp.int32),
        pltpu.VMEM((_WG, _CH), jnp.int32),
        pltpu.VMEM((_WG, _CH), jnp.float32),
        pltpu.VMEM((_NP,), jnp.float32),
        pltpu.VMEM((_CH, _D), jnp.float32),
        pltpu.VMEM((_CH,), jnp.float32),
        pltpu.VMEM_SHARED((_NP, _D), jnp.float32),
        pltpu.SemaphoreType.DMA,
    ],
)
def _msg_sc(xw, srcm, dstm, ewm, dinv_h, out,
            srcv, dstv, ewv, dinv_v, rows, norm, acc, gsem):
    c = lax.axis_index("c")
    s = lax.axis_index("s")
    w = c * 16 + s

    def zrow(r, carry):
        for i in range(8):
            rows[r, pl.ds(16 * i, 16)] = _z16()
        return carry

    lax.fori_loop(0, _CH, zrow, 0)
    for k in range(5):
        pltpu.sync_copy(rows, acc.at[pl.ds(s * 640 + k * 128, 128)])
    plsc.subcore_barrier()

    pltpu.sync_copy(dinv_h, dinv_v)

    def group(g, carry):
        pltpu.sync_copy(srcm.at[w, pl.ds(g * _WG, _WG)], srcv)
        pltpu.sync_copy(dstm.at[w, pl.ds(g * _WG, _WG)], dstv)
        pltpu.sync_copy(ewm.at[w, pl.ds(g * _WG, _WG)], ewv)
        nb = jnp.minimum(_WG, _NCH - g * _WG)

        def chunk(j, carry1):
            for f in range(8):
                si = srcv[j, pl.ds(16 * f, 16)]
                di = dstv[j, pl.ds(16 * f, 16)]
                gs = plsc.load_gather(dinv_v, [si])
                gd = plsc.load_gather(dinv_v, [di])
                norm[pl.ds(16 * f, 16)] = gs * ewv[j, pl.ds(16 * f, 16)] * gd

            pltpu.async_copy(xw.at[srcv.at[j]], rows, gsem).wait()

            def scale(r, carry2):
                sp = plsc.load_gather(norm, [jnp.full((16,), r, jnp.int32)])
                for f in range(8):
                    rows[r, pl.ds(16 * f, 16)] = rows[r, pl.ds(16 * f, 16)] * sp
                return carry2

            lax.fori_loop(0, _CH, scale, 0)
            pltpu.sync_copy(rows, acc.at[dstv.at[j]], add=True)
            return carry1

        lax.fori_loop(0, nb, chunk, 0)
        return carry

    lax.fori_loop(0, (_NCH + _WG - 1) // _WG, group, 0)

    plsc.subcore_barrier()
    pltpu.sync_copy(acc.at[pl.ds(s * 640, 640)], out.at[c, pl.ds(s * 640, 640)])


def _mm1_body(x_ref, w_ref, o_ref):
    o_ref[...] = jnp.dot(x_ref[...], w_ref[...],
                         preferred_element_type=jnp.float32)


_mm1 = pl.pallas_call(
    _mm1_body,
    grid=(_NG,),
    in_specs=[
        pl.BlockSpec((_NB, _D), lambda i: (i, 0)),
        pl.BlockSpec((_D, _D), lambda i: (0, 0)),
    ],
    out_specs=pl.BlockSpec((_NB, _D), lambda i: (i, 0)),
    out_shape=jax.ShapeDtypeStruct((_NP, _D), jnp.float32),
)


def _dinv_body(dp_ref, o_ref):
    deg = dp_ref[0:1, :] + dp_ref[1:2, :]
    o_ref[...] = jnp.where(deg > 0, lax.rsqrt(jnp.maximum(deg, 1e-12)), 0.0)


_dinvk = pl.pallas_call(
    _dinv_body,
    out_shape=jax.ShapeDtypeStruct((1, _NP), jnp.float32),
)


def _mm2_body(p_ref, bias_ref, w_ref, o_ref):
    h = p_ref[0] + p_ref[1] + bias_ref[...]
    o_ref[...] = jnp.dot(h, w_ref[...], preferred_element_type=jnp.float32)


_mm2 = pl.pallas_call(
    _mm2_body,
    grid=(_NG,),
    in_specs=[
        pl.BlockSpec((2, _NB, _D), lambda i: (0, i, 0)),
        pl.BlockSpec((1, _D), lambda i: (0, 0)),
        pl.BlockSpec((_D, _D), lambda i: (0, 0)),
    ],
    out_specs=pl.BlockSpec((_NB, _D), lambda i: (i, 0)),
    out_shape=jax.ShapeDtypeStruct((_NP, _D), jnp.float32),
)


def _fin_body(q_ref, bias_ref, o_ref):
    o_ref[...] = q_ref[0] + q_ref[1] + bias_ref[...]


_fin = pl.pallas_call(
    _fin_body,
    grid=(_NG,),
    in_specs=[
        pl.BlockSpec((2, _NB, _D), lambda i: (0, i, 0)),
        pl.BlockSpec((1, _D), lambda i: (0, 0)),
    ],
    out_specs=pl.BlockSpec((_NB, _D), lambda i: (i, 0)),
    out_shape=jax.ShapeDtypeStruct((_NP, _D), jnp.float32),
)


def kernel(x, edge_index, edge_attr, y, W1, b1, W2, b2):
    del y
    src = edge_index[0]
    dst = edge_index[1]
    ew = edge_attr[:, 2]
    loop = jnp.arange(_N, dtype=src.dtype)
    padn = _EP - (_E + _N)
    padidx = jnp.arange(padn, dtype=src.dtype) % _N

    def worker_view(flat):
        return flat.reshape(_WT, 32, _CH).transpose(1, 0, 2)

    src2 = worker_view(jnp.concatenate([src, loop, padidx]))
    dst2 = worker_view(jnp.concatenate([dst, loop, padidx]))
    ew2 = worker_view(jnp.concatenate([ew, jnp.ones((_N,), x.dtype),
                                       jnp.zeros((padn,), x.dtype)]))

    x_p = jnp.pad(x, ((0, _NP - _N), (0, 0)))

    xw1 = _mm1(x_p, W1)
    degp = _deg_sc(dst2, ew2)
    dinv = _dinvk(degp).reshape(_NP)

    p = _msg_sc(xw1, src2, dst2, ew2, dinv)
    xw2 = _mm2(p, b1.reshape(1, _D), W2)
    q = _msg_sc(xw2, src2, dst2, ew2, dinv)
    out_p = _fin(q, b2.reshape(1, _D))
    return out_p[:_N]
